# Initial kernel scaffold; baseline (speedup 1.0000x reference)
import jax
import jax.numpy as jnp
from jax.experimental import pallas as pl


def kernel(x, adj, W1l, b1, W1r, W2l, b2, W2r):
    raise NotImplementedError("write your pallas kernel here")


# speedup vs baseline: 7.6189x; 7.6189x over previous
import functools

import jax
import jax.numpy as jnp
from jax import lax
from jax.experimental import pallas as pl
from jax.experimental.pallas import tpu as pltpu
from jax.experimental.pallas import tpu_sc as plsc

N = 10000
E = 320000
D = 128
CW = 16
NC = 2
NS = 16
NW = NC * NS
L = 16
EPW = E // NW
C = 80
NCHUNK = EPW // C
SSC = 5
ICH = NCHUNK // SSC
NPAD = 10240
RPS = NPAD // NS


def _make_sc_agg(with_counts: bool):
    mesh = plsc.VectorSubcoreMesh(core_axis_name="c", subcore_axis_name="s")
    out_type = [jax.ShapeDtypeStruct((NC, NPAD, D), jnp.float32)]
    scratch = [
        pltpu.VMEM_SHARED((NPAD, D), jnp.float32),
        pltpu.VMEM((ICH, C), jnp.int32),
        pltpu.VMEM((ICH, C), jnp.int32),
        pltpu.VMEM((C, D), jnp.float32),
        pltpu.SemaphoreType.DMA,
    ]
    if with_counts:
        out_type.append(jax.ShapeDtypeStruct((NW, NPAD), jnp.float32))
        scratch.append(pltpu.VMEM((NPAD,), jnp.float32))

    @functools.partial(pl.kernel, mesh=mesh, out_type=out_type,
                       scratch_types=scratch,
                       compiler_params=pltpu.CompilerParams(
                           needs_layout_passes=False))
    def sc_agg(*refs):
        if with_counts:
            (h_hbm, srcr, dstr, zrow,
             out_hbm, cnt_hbm, accum, srcv, dstv, rows, sem, cntloc) = refs
        else:
            (h_hbm, srcr, dstr, zrow,
             out_hbm, accum, srcv, dstv, rows, sem) = refs
        c = lax.axis_index("c")
        s = lax.axis_index("s")
        wid = s * NC + c

        pltpu.sync_copy(zrow.at[pl.ds(s * RPS, RPS)], accum.at[pl.ds(s * RPS, RPS)])
        if with_counts:
            def zc(i, carry):
                cntloc[pl.ds(i * L, L)] = jnp.zeros((L,), jnp.float32)
                return carry
            lax.fori_loop(0, NPAD // L, zc, 0)
        plsc.subcore_barrier()

        def body(i, carry):
            cp = pltpu.async_copy(h_hbm.at[srcv.at[i]], rows, sem)
            if with_counts:
                for j in range(C // L):
                    idx = dstv[i, pl.ds(j * L, L)]
                    cnts, last = plsc.scan_count(idx)
                    plsc.addupdate_scatter(
                        cntloc, [idx], cnts.astype(jnp.float32), mask=last)
            cp.wait()
            pltpu.sync_copy(rows, accum.at[dstv.at[i]], add=True)
            return carry

        def outer(o, carry):
            pltpu.sync_copy(srcr.at[wid, o], srcv)
            pltpu.sync_copy(dstr.at[wid, o], dstv)
            lax.fori_loop(0, ICH, body, 0)
            return carry

        lax.fori_loop(0, SSC, outer, 0)
        plsc.subcore_barrier()

        pltpu.sync_copy(accum.at[pl.ds(s * RPS, RPS)],
                        out_hbm.at[c, pl.ds(s * RPS, RPS)])
        if with_counts:
            pltpu.sync_copy(cntloc, cnt_hbm.at[wid])

    return sc_agg


_sc_agg_l1 = _make_sc_agg(True)
_sc_agg_l2 = _make_sc_agg(False)

BLK = 2000


def _tc1_body(parts_ref, cnt_ref, h_ref, wl_ref, b_ref, wr_ref, out_ref, invc_ref):
    cnt = jnp.sum(cnt_ref[...], axis=1, keepdims=True)
    invc = 1.0 / jnp.maximum(cnt, 1.0)
    mean = (parts_ref[0] + parts_ref[1]) * invc
    acc = (jnp.dot(mean, wl_ref[...], preferred_element_type=jnp.float32)
           + b_ref[...]
           + jnp.dot(h_ref[...], wr_ref[...], preferred_element_type=jnp.float32))
    out_ref[...] = jnp.maximum(acc, 0.0)
    invc_ref[...] = jnp.broadcast_to(invc, (BLK, CW))


def _tc2_body(parts_ref, invc_ref, h_ref, wl_ref, b_ref, wr_ref, out_ref):
    invc = invc_ref[:, 0:1]
    mean = (parts_ref[0] + parts_ref[1]) * invc
    acc = (jnp.dot(mean, wl_ref[...], preferred_element_type=jnp.float32)
           + b_ref[...]
           + jnp.dot(h_ref[...], wr_ref[...], preferred_element_type=jnp.float32))
    out_ref[...] = jnp.maximum(acc, 0.0)


def _tc_layer1(parts, cntT, h, Wl, b, Wr):
    return pl.pallas_call(
        _tc1_body,
        grid=(N // BLK,),
        in_specs=[
            pl.BlockSpec((NC, BLK, D), lambda i: (0, i, 0)),
            pl.BlockSpec((BLK, NW), lambda i: (i, 0)),
            pl.BlockSpec((BLK, D), lambda i: (i, 0)),
            pl.BlockSpec((D, D), lambda i: (0, 0)),
            pl.BlockSpec((1, D), lambda i: (0, 0)),
            pl.BlockSpec((D, D), lambda i: (0, 0)),
        ],
        out_specs=[
            pl.BlockSpec((BLK, D), lambda i: (i, 0)),
            pl.BlockSpec((BLK, CW), lambda i: (i, 0)),
        ],
        out_shape=[
            jax.ShapeDtypeStruct((N, D), jnp.float32),
            jax.ShapeDtypeStruct((N, CW), jnp.float32),
        ],
    )(parts, cntT, h, Wl, b, Wr)


def _tc_layer2(parts, invc, h, Wl, b, Wr):
    return pl.pallas_call(
        _tc2_body,
        grid=(N // BLK,),
        in_specs=[
            pl.BlockSpec((NC, BLK, D), lambda i: (0, i, 0)),
            pl.BlockSpec((BLK, CW), lambda i: (i, 0)),
            pl.BlockSpec((BLK, D), lambda i: (i, 0)),
            pl.BlockSpec((D, D), lambda i: (0, 0)),
            pl.BlockSpec((1, D), lambda i: (0, 0)),
            pl.BlockSpec((D, D), lambda i: (0, 0)),
        ],
        out_specs=pl.BlockSpec((BLK, D), lambda i: (i, 0)),
        out_shape=jax.ShapeDtypeStruct((N, D), jnp.float32),
    )(parts, invc, h, Wl, b, Wr)


def kernel(x, adj, W1l, b1, W1r, W2l, b2, W2r):
    src = adj[0].astype(jnp.int32).reshape(NW, SSC, ICH, C)
    dst = adj[1].astype(jnp.int32).reshape(NW, SSC, ICH, C)
    zrow = jnp.zeros((NPAD, D), jnp.float32)

    parts1, cnt = _sc_agg_l1(x, src, dst, zrow)
    h1, invc = _tc_layer1(parts1[:, :N], cnt.T[:N], x, W1l, b1.reshape(1, D), W1r)
    (parts2,) = _sc_agg_l2(h1, src, dst, zrow)
    return _tc_layer2(parts2[:, :N], invc, h1, W2l, b2.reshape(1, D), W2r)


# speedup vs baseline: 11.4013x; 1.4965x over previous
import functools

import jax
import jax.numpy as jnp
from jax import lax
from jax.experimental import pallas as pl
from jax.experimental.pallas import tpu as pltpu
from jax.experimental.pallas import tpu_sc as plsc

N = 10000
E = 320000
D = 128
CW = 16
NC = 2
NS = 16
NW = NC * NS
L = 16
EPW = E // NW
C = 80
NCHUNK = EPW // C
SSC = 5
ICH = NCHUNK // SSC
NPAD = 10240
RPS = NPAD // NS


def _make_sc_agg(with_counts: bool):
    mesh = plsc.VectorSubcoreMesh(core_axis_name="c", subcore_axis_name="s")
    out_type = [jax.ShapeDtypeStruct((NC, NPAD, D), jnp.float32)]
    scratch = [
        pltpu.VMEM_SHARED((NPAD, D), jnp.float32),
        pltpu.VMEM((ICH, C), jnp.int32),
        pltpu.VMEM((ICH, C), jnp.int32),
        pltpu.VMEM((C, D), jnp.float32),
        pltpu.VMEM((C, D), jnp.float32),
        pltpu.SemaphoreType.DMA,
        pltpu.SemaphoreType.DMA,
    ]
    if with_counts:
        out_type.append(jax.ShapeDtypeStruct((NW, NPAD), jnp.float32))
        scratch.append(pltpu.VMEM((NPAD,), jnp.float32))

    @functools.partial(pl.kernel, mesh=mesh, out_type=out_type,
                       scratch_types=scratch,
                       compiler_params=pltpu.CompilerParams(
                           needs_layout_passes=False))
    def sc_agg(*refs):
        if with_counts:
            (h_hbm, srcr, dstr, zrow,
             out_hbm, cnt_hbm, accum, srcv, dstv,
             rows0, rows1, sem0, sem1, cntloc) = refs
        else:
            (h_hbm, srcr, dstr, zrow,
             out_hbm, accum, srcv, dstv, rows0, rows1, sem0, sem1) = refs
        c = lax.axis_index("c")
        s = lax.axis_index("s")
        wid = s * NC + c

        pltpu.sync_copy(zrow.at[pl.ds(s * RPS, RPS)], accum.at[pl.ds(s * RPS, RPS)])
        if with_counts:
            def zc(i, carry):
                cntloc[pl.ds(i * L, L)] = jnp.zeros((L,), jnp.float32)
                return carry
            lax.fori_loop(0, NPAD // L, zc, 0)
        plsc.subcore_barrier()

        def hist(i):
            if with_counts:
                for j in range(C // L):
                    idx = dstv[i, pl.ds(j * L, L)]
                    cnts, last = plsc.scan_count(idx)
                    plsc.addupdate_scatter(
                        cntloc, [idx], cnts.astype(jnp.float32), mask=last)

        def gather(i, rows, sem):
            return pltpu.async_copy(h_hbm.at[srcv.at[i]], rows, sem)

        def scatter(i, rows):
            pltpu.sync_copy(rows, accum.at[dstv.at[i]], add=True)

        def outer(o, carry):
            pltpu.sync_copy(srcr.at[wid, o], srcv)
            pltpu.sync_copy(dstr.at[wid, o], dstv)
            gather(0, rows0, sem0)

            def pair(k, carry):
                i0 = k * 2
                gather(i0 + 1, rows1, sem1)
                hist(i0)
                pltpu.make_async_copy(h_hbm.at[srcv.at[i0]], rows0, sem0).wait()
                scatter(i0, rows0)
                gather(i0 + 2, rows0, sem0)
                hist(i0 + 1)
                pltpu.make_async_copy(h_hbm.at[srcv.at[i0 + 1]], rows1, sem1).wait()
                scatter(i0 + 1, rows1)
                return carry

            lax.fori_loop(0, (ICH - 1) // 2, pair, 0)
            hist(ICH - 1)
            pltpu.make_async_copy(h_hbm.at[srcv.at[ICH - 1]], rows0, sem0).wait()
            scatter(ICH - 1, rows0)
            return carry

        lax.fori_loop(0, SSC, outer, 0)
        plsc.subcore_barrier()

        pltpu.sync_copy(accum.at[pl.ds(s * RPS, RPS)],
                        out_hbm.at[c, pl.ds(s * RPS, RPS)])
        if with_counts:
            pltpu.sync_copy(cntloc, cnt_hbm.at[wid])

    return sc_agg


_sc_agg_l1 = _make_sc_agg(True)
_sc_agg_l2 = _make_sc_agg(False)

BLK = 2000


def _tc1_body(parts_ref, cnt_ref, h_ref, wl_ref, b_ref, wr_ref, out_ref, invc_ref):
    cnt = jnp.sum(cnt_ref[...], axis=1, keepdims=True)
    invc = 1.0 / jnp.maximum(cnt, 1.0)
    mean = (parts_ref[0] + parts_ref[1]) * invc
    acc = (jnp.dot(mean, wl_ref[...], preferred_element_type=jnp.float32)
           + b_ref[...]
           + jnp.dot(h_ref[...], wr_ref[...], preferred_element_type=jnp.float32))
    out_ref[...] = jnp.maximum(acc, 0.0)
    invc_ref[...] = jnp.broadcast_to(invc, (BLK, CW))


def _tc2_body(parts_ref, invc_ref, h_ref, wl_ref, b_ref, wr_ref, out_ref):
    invc = invc_ref[:, 0:1]
    mean = (parts_ref[0] + parts_ref[1]) * invc
    acc = (jnp.dot(mean, wl_ref[...], preferred_element_type=jnp.float32)
           + b_ref[...]
           + jnp.dot(h_ref[...], wr_ref[...], preferred_element_type=jnp.float32))
    out_ref[...] = jnp.maximum(acc, 0.0)


def _tc_layer1(parts, cntT, h, Wl, b, Wr):
    return pl.pallas_call(
        _tc1_body,
        grid=(N // BLK,),
        in_specs=[
            pl.BlockSpec((NC, BLK, D), lambda i: (0, i, 0)),
            pl.BlockSpec((BLK, NW), lambda i: (i, 0)),
            pl.BlockSpec((BLK, D), lambda i: (i, 0)),
            pl.BlockSpec((D, D), lambda i: (0, 0)),
            pl.BlockSpec((1, D), lambda i: (0, 0)),
            pl.BlockSpec((D, D), lambda i: (0, 0)),
        ],
        out_specs=[
            pl.BlockSpec((BLK, D), lambda i: (i, 0)),
            pl.BlockSpec((BLK, CW), lambda i: (i, 0)),
        ],
        out_shape=[
            jax.ShapeDtypeStruct((N, D), jnp.float32),
            jax.ShapeDtypeStruct((N, CW), jnp.float32),
        ],
    )(parts, cntT, h, Wl, b, Wr)


def _tc_layer2(parts, invc, h, Wl, b, Wr):
    return pl.pallas_call(
        _tc2_body,
        grid=(N // BLK,),
        in_specs=[
            pl.BlockSpec((NC, BLK, D), lambda i: (0, i, 0)),
            pl.BlockSpec((BLK, CW), lambda i: (i, 0)),
            pl.BlockSpec((BLK, D), lambda i: (i, 0)),
            pl.BlockSpec((D, D), lambda i: (0, 0)),
            pl.BlockSpec((1, D), lambda i: (0, 0)),
            pl.BlockSpec((D, D), lambda i: (0, 0)),
        ],
        out_specs=pl.BlockSpec((BLK, D), lambda i: (i, 0)),
        out_shape=jax.ShapeDtypeStruct((N, D), jnp.float32),
    )(parts, invc, h, Wl, b, Wr)


def kernel(x, adj, W1l, b1, W1r, W2l, b2, W2r):
    src = adj[0].astype(jnp.int32).reshape(NW, SSC, ICH, C)
    dst = adj[1].astype(jnp.int32).reshape(NW, SSC, ICH, C)
    zrow = jnp.zeros((NPAD, D), jnp.float32)

    parts1, cnt = _sc_agg_l1(x, src, dst, zrow)
    h1, invc = _tc_layer1(parts1[:, :N], cnt.T[:N], x, W1l, b1.reshape(1, D), W1r)
    (parts2,) = _sc_agg_l2(h1, src, dst, zrow)
    return _tc_layer2(parts2[:, :N], invc, h1, W2l, b2.reshape(1, D), W2r)


# speedup vs baseline: 13.5134x; 1.1852x over previous
import functools

import jax
import jax.numpy as jnp
from jax import lax
from jax.experimental import pallas as pl
from jax.experimental.pallas import tpu as pltpu
from jax.experimental.pallas import tpu_sc as plsc

N = 10000
E = 320000
D = 128
CW = 16
NC = 2
NS = 16
NW = NC * NS
L = 16
EPW = E // NW
C = 80
NCHUNK = EPW // C
SSC = 5
ICH = NCHUNK // SSC
NPAD = 10240
RPS = NPAD // NS


def _make_sc_agg(with_counts: bool):
    mesh = plsc.VectorSubcoreMesh(core_axis_name="c", subcore_axis_name="s")
    out_type = [jax.ShapeDtypeStruct((NC, NPAD, D), jnp.float32)]
    scratch = [
        pltpu.VMEM_SHARED((NPAD, D), jnp.float32),
        pltpu.VMEM((ICH, C), jnp.int32),
        pltpu.VMEM((ICH, C), jnp.int32),
        pltpu.VMEM((C, D), jnp.float32),
        pltpu.VMEM((C, D), jnp.float32),
        pltpu.VMEM((C, D), jnp.float32),
        pltpu.SemaphoreType.DMA,
        pltpu.SemaphoreType.DMA,
        pltpu.SemaphoreType.DMA,
    ]
    if with_counts:
        out_type.append(jax.ShapeDtypeStruct((NW, NPAD), jnp.float32))
        scratch.append(pltpu.VMEM((NPAD,), jnp.float32))

    @functools.partial(pl.kernel, mesh=mesh, out_type=out_type,
                       scratch_types=scratch,
                       compiler_params=pltpu.CompilerParams(
                           needs_layout_passes=False))
    def sc_agg(*refs):
        if with_counts:
            (h_hbm, srcr, dstr, zrow,
             out_hbm, cnt_hbm, accum, srcv, dstv,
             rows0, rows1, rows2, sem0, sem1, sem2, cntloc) = refs
        else:
            (h_hbm, srcr, dstr, zrow,
             out_hbm, accum, srcv, dstv,
             rows0, rows1, rows2, sem0, sem1, sem2) = refs
        bufs = [(rows0, sem0), (rows1, sem1), (rows2, sem2)]
        c = lax.axis_index("c")
        s = lax.axis_index("s")
        wid = s * NC + c

        pltpu.sync_copy(zrow.at[pl.ds(s * RPS, RPS)], accum.at[pl.ds(s * RPS, RPS)])
        if with_counts:
            def zc(i, carry):
                cntloc[pl.ds(i * L, L)] = jnp.zeros((L,), jnp.float32)
                return carry
            lax.fori_loop(0, NPAD // L, zc, 0)
        plsc.subcore_barrier()

        def hist(i):
            if with_counts:
                for j in range(C // L):
                    idx = dstv[i, pl.ds(j * L, L)]
                    cnts, last = plsc.scan_count(idx)
                    plsc.addupdate_scatter(
                        cntloc, [idx], cnts.astype(jnp.float32), mask=last)

        def gather(i, b):
            rows, sem = bufs[b]
            pltpu.async_copy(h_hbm.at[srcv.at[i]], rows, sem)

        def wait_scatter(i, b):
            rows, sem = bufs[b]
            pltpu.make_async_copy(h_hbm.at[srcv.at[i]], rows, sem).wait()
            pltpu.sync_copy(rows, accum.at[dstv.at[i]], add=True)

        M = (ICH - 2) // 3
        LEFT = ICH - 3 * M

        def outer(o, carry):
            pltpu.sync_copy(srcr.at[wid, o], srcv)
            pltpu.sync_copy(dstr.at[wid, o], dstv)
            gather(0, 0)
            gather(1, 1)

            def triple(k, carry):
                i0 = k * 3
                for t in range(3):
                    gather(i0 + t + 2, (t + 2) % 3)
                    hist(i0 + t)
                    wait_scatter(i0 + t, t)
                return carry

            lax.fori_loop(0, M, triple, 0)
            base = 3 * M
            for t in range(LEFT):
                c = base + t
                if c + 2 < ICH:
                    gather(c + 2, (c + 2) % 3)
                hist(c)
                wait_scatter(c, c % 3)
            return carry

        lax.fori_loop(0, SSC, outer, 0)
        plsc.subcore_barrier()

        pltpu.sync_copy(accum.at[pl.ds(s * RPS, RPS)],
                        out_hbm.at[c, pl.ds(s * RPS, RPS)])
        if with_counts:
            pltpu.sync_copy(cntloc, cnt_hbm.at[wid])

    return sc_agg


_sc_agg_l1 = _make_sc_agg(True)
_sc_agg_l2 = _make_sc_agg(False)

BLK = 2000


def _tc1_body(parts_ref, cnt_ref, h_ref, wl_ref, b_ref, wr_ref, out_ref, invc_ref):
    cnt = jnp.sum(cnt_ref[...], axis=1, keepdims=True)
    invc = 1.0 / jnp.maximum(cnt, 1.0)
    mean = (parts_ref[0] + parts_ref[1]) * invc
    acc = (jnp.dot(mean, wl_ref[...], preferred_element_type=jnp.float32)
           + b_ref[...]
           + jnp.dot(h_ref[...], wr_ref[...], preferred_element_type=jnp.float32))
    out_ref[...] = jnp.maximum(acc, 0.0)
    invc_ref[...] = jnp.broadcast_to(invc, (BLK, CW))


def _tc2_body(parts_ref, invc_ref, h_ref, wl_ref, b_ref, wr_ref, out_ref):
    invc = invc_ref[:, 0:1]
    mean = (parts_ref[0] + parts_ref[1]) * invc
    acc = (jnp.dot(mean, wl_ref[...], preferred_element_type=jnp.float32)
           + b_ref[...]
           + jnp.dot(h_ref[...], wr_ref[...], preferred_element_type=jnp.float32))
    out_ref[...] = jnp.maximum(acc, 0.0)


def _tc_layer1(parts, cntT, h, Wl, b, Wr):
    return pl.pallas_call(
        _tc1_body,
        grid=(N // BLK,),
        in_specs=[
            pl.BlockSpec((NC, BLK, D), lambda i: (0, i, 0)),
            pl.BlockSpec((BLK, NW), lambda i: (i, 0)),
            pl.BlockSpec((BLK, D), lambda i: (i, 0)),
            pl.BlockSpec((D, D), lambda i: (0, 0)),
            pl.BlockSpec((1, D), lambda i: (0, 0)),
            pl.BlockSpec((D, D), lambda i: (0, 0)),
        ],
        out_specs=[
            pl.BlockSpec((BLK, D), lambda i: (i, 0)),
            pl.BlockSpec((BLK, CW), lambda i: (i, 0)),
        ],
        out_shape=[
            jax.ShapeDtypeStruct((N, D), jnp.float32),
            jax.ShapeDtypeStruct((N, CW), jnp.float32),
        ],
    )(parts, cntT, h, Wl, b, Wr)


def _tc_layer2(parts, invc, h, Wl, b, Wr):
    return pl.pallas_call(
        _tc2_body,
        grid=(N // BLK,),
        in_specs=[
            pl.BlockSpec((NC, BLK, D), lambda i: (0, i, 0)),
            pl.BlockSpec((BLK, CW), lambda i: (i, 0)),
            pl.BlockSpec((BLK, D), lambda i: (i, 0)),
            pl.BlockSpec((D, D), lambda i: (0, 0)),
            pl.BlockSpec((1, D), lambda i: (0, 0)),
            pl.BlockSpec((D, D), lambda i: (0, 0)),
        ],
        out_specs=pl.BlockSpec((BLK, D), lambda i: (i, 0)),
        out_shape=jax.ShapeDtypeStruct((N, D), jnp.float32),
    )(parts, invc, h, Wl, b, Wr)


def kernel(x, adj, W1l, b1, W1r, W2l, b2, W2r):
    src = adj[0].astype(jnp.int32).reshape(NW, SSC, ICH, C)
    dst = adj[1].astype(jnp.int32).reshape(NW, SSC, ICH, C)
    zrow = jnp.zeros((NPAD, D), jnp.float32)

    parts1, cnt = _sc_agg_l1(x, src, dst, zrow)
    h1, invc = _tc_layer1(parts1, cnt.T, x, W1l, b1.reshape(1, D), W1r)
    (parts2,) = _sc_agg_l2(h1, src, dst, zrow)
    return _tc_layer2(parts2, invc, h1, W2l, b2.reshape(1, D), W2r)


# speedup vs baseline: 13.5578x; 1.0033x over previous
import functools

import jax
import jax.numpy as jnp
from jax import lax
from jax.experimental import pallas as pl
from jax.experimental.pallas import tpu as pltpu
from jax.experimental.pallas import tpu_sc as plsc

N = 10000
E = 320000
D = 128
CW = 16
NC = 2
NS = 16
NW = NC * NS
L = 16
EPW = E // NW
C = 80
NCHUNK = EPW // C
SSC = 5
ICH = NCHUNK // SSC
NPAD = 10240
RPS = NPAD // NS


def _make_sc_agg(with_counts: bool):
    mesh = plsc.VectorSubcoreMesh(core_axis_name="c", subcore_axis_name="s")
    out_type = [jax.ShapeDtypeStruct((NC, NPAD, D), jnp.float32)]
    scratch = [
        pltpu.VMEM_SHARED((NPAD, D), jnp.float32),
        pltpu.VMEM((ICH, C), jnp.int32),
        pltpu.VMEM((ICH, C), jnp.int32),
        pltpu.VMEM((C, D), jnp.float32),
        pltpu.VMEM((C, D), jnp.float32),
        pltpu.VMEM((C, D), jnp.float32),
        pltpu.SemaphoreType.DMA,
        pltpu.SemaphoreType.DMA,
        pltpu.SemaphoreType.DMA,
    ]
    if with_counts:
        out_type.append(jax.ShapeDtypeStruct((NW, NPAD), jnp.float32))
        scratch.append(pltpu.VMEM((NPAD,), jnp.float32))

    @functools.partial(pl.kernel, mesh=mesh, out_type=out_type,
                       scratch_types=scratch,
                       compiler_params=pltpu.CompilerParams(
                           needs_layout_passes=False))
    def sc_agg(*refs):
        if with_counts:
            (h_hbm, srcr, dstr, zrow,
             out_hbm, cnt_hbm, accum, srcv, dstv,
             rows0, rows1, rows2, sem0, sem1, sem2, cntloc) = refs
        else:
            (h_hbm, srcr, dstr, zrow,
             out_hbm, accum, srcv, dstv,
             rows0, rows1, rows2, sem0, sem1, sem2) = refs
        bufs = [(rows0, sem0), (rows1, sem1), (rows2, sem2)]
        c = lax.axis_index("c")
        s = lax.axis_index("s")
        wid = s * NC + c


        def hist(i):
            if with_counts:
                for j in range(C // L):
                    idx = dstv[i, pl.ds(j * L, L)]
                    cnts, last = plsc.scan_count(idx)
                    plsc.addupdate_scatter(
                        cntloc, [idx], cnts.astype(jnp.float32), mask=last)

        def gather(i, b):
            rows, sem = bufs[b]
            pltpu.async_copy(h_hbm.at[srcv.at[i]], rows, sem)

        def wait_scatter(i, b):
            rows, sem = bufs[b]
            pltpu.make_async_copy(h_hbm.at[srcv.at[i]], rows, sem).wait()
            pltpu.sync_copy(rows, accum.at[dstv.at[i]], add=True)

        M = (ICH - 2) // 3
        LEFT = ICH - 3 * M

        def triple(k, carry):
            i0 = k * 3
            for t in range(3):
                gather(i0 + t + 2, (t + 2) % 3)
                hist(i0 + t)
                wait_scatter(i0 + t, t)
            return carry

        for o in range(SSC):
            pltpu.sync_copy(srcr.at[wid, o], srcv)
            pltpu.sync_copy(dstr.at[wid, o], dstv)
            gather(0, 0)
            gather(1, 1)
            if o == 0:
                pltpu.sync_copy(zrow.at[pl.ds(s * RPS, RPS)],
                                accum.at[pl.ds(s * RPS, RPS)])
                if with_counts:
                    def zc(i, carry):
                        cntloc[pl.ds(i * L, L)] = jnp.zeros((L,), jnp.float32)
                        return carry
                    lax.fori_loop(0, NPAD // L, zc, 0)
                plsc.subcore_barrier()
            lax.fori_loop(0, M, triple, 0)
            base = 3 * M
            for t in range(LEFT):
                c = base + t
                if c + 2 < ICH:
                    gather(c + 2, (c + 2) % 3)
                hist(c)
                wait_scatter(c, c % 3)
        plsc.subcore_barrier()

        pltpu.sync_copy(accum.at[pl.ds(s * RPS, RPS)],
                        out_hbm.at[c, pl.ds(s * RPS, RPS)])
        if with_counts:
            pltpu.sync_copy(cntloc, cnt_hbm.at[wid])

    return sc_agg


_sc_agg_l1 = _make_sc_agg(True)
_sc_agg_l2 = _make_sc_agg(False)

BLK = 2000


def _tc_pre_body(h_ref, wr_ref, b_ref, out_ref):
    out_ref[...] = (jnp.dot(h_ref[...], wr_ref[...],
                            preferred_element_type=jnp.float32) + b_ref[...])


def _tc1_body(parts_ref, cnt_ref, pre_ref, wl_ref, out_ref, invc_ref):
    cnt = jnp.sum(cnt_ref[...], axis=1, keepdims=True)
    invc = 1.0 / jnp.maximum(cnt, 1.0)
    mean = (parts_ref[0] + parts_ref[1]) * invc
    acc = (jnp.dot(mean, wl_ref[...], preferred_element_type=jnp.float32)
           + pre_ref[...])
    out_ref[...] = jnp.maximum(acc, 0.0)
    invc_ref[...] = jnp.broadcast_to(invc, (BLK, CW))


def _tc2_body(parts_ref, invc_ref, pre_ref, wl_ref, out_ref):
    invc = invc_ref[:, 0:1]
    mean = (parts_ref[0] + parts_ref[1]) * invc
    acc = (jnp.dot(mean, wl_ref[...], preferred_element_type=jnp.float32)
           + pre_ref[...])
    out_ref[...] = jnp.maximum(acc, 0.0)


def _tc_pre(h, Wr, b):
    return pl.pallas_call(
        _tc_pre_body,
        grid=(N // BLK,),
        in_specs=[
            pl.BlockSpec((BLK, D), lambda i: (i, 0)),
            pl.BlockSpec((D, D), lambda i: (0, 0)),
            pl.BlockSpec((1, D), lambda i: (0, 0)),
        ],
        out_specs=pl.BlockSpec((BLK, D), lambda i: (i, 0)),
        out_shape=jax.ShapeDtypeStruct((N, D), jnp.float32),
    )(h, Wr, b)


def _tc_layer1(parts, cntT, pre, Wl):
    return pl.pallas_call(
        _tc1_body,
        grid=(N // BLK,),
        in_specs=[
            pl.BlockSpec((NC, BLK, D), lambda i: (0, i, 0)),
            pl.BlockSpec((BLK, NW), lambda i: (i, 0)),
            pl.BlockSpec((BLK, D), lambda i: (i, 0)),
            pl.BlockSpec((D, D), lambda i: (0, 0)),
        ],
        out_specs=[
            pl.BlockSpec((BLK, D), lambda i: (i, 0)),
            pl.BlockSpec((BLK, CW), lambda i: (i, 0)),
        ],
        out_shape=[
            jax.ShapeDtypeStruct((N, D), jnp.float32),
            jax.ShapeDtypeStruct((N, CW), jnp.float32),
        ],
    )(parts, cntT, pre, Wl)


def _tc_layer2(parts, invc, pre, Wl):
    return pl.pallas_call(
        _tc2_body,
        grid=(N // BLK,),
        in_specs=[
            pl.BlockSpec((NC, BLK, D), lambda i: (0, i, 0)),
            pl.BlockSpec((BLK, CW), lambda i: (i, 0)),
            pl.BlockSpec((BLK, D), lambda i: (i, 0)),
            pl.BlockSpec((D, D), lambda i: (0, 0)),
        ],
        out_specs=pl.BlockSpec((BLK, D), lambda i: (i, 0)),
        out_shape=jax.ShapeDtypeStruct((N, D), jnp.float32),
    )(parts, invc, pre, Wl)


def kernel(x, adj, W1l, b1, W1r, W2l, b2, W2r):
    src = adj[0].astype(jnp.int32).reshape(NW, SSC, ICH, C)
    dst = adj[1].astype(jnp.int32).reshape(NW, SSC, ICH, C)
    zrow = jnp.zeros((NPAD, D), jnp.float32)

    pre1 = _tc_pre(x, W1r, b1.reshape(1, D))
    parts1, cnt = _sc_agg_l1(x, src, dst, zrow)
    h1, invc = _tc_layer1(parts1, cnt.T, pre1, W1l)
    pre2 = _tc_pre(h1, W2r, b2.reshape(1, D))
    (parts2,) = _sc_agg_l2(h1, src, dst, zrow)
    return _tc_layer2(parts2, invc, pre2, W2l)
